# Initial kernel scaffold; baseline (speedup 1.0000x reference)
import jax
import jax.numpy as jnp
from jax.experimental import pallas as pl


def kernel(x, mask):
    raise NotImplementedError("write your pallas kernel here")


# speedup vs baseline: 2.1848x; 2.1848x over previous
import jax
import jax.numpy as jnp
from jax.experimental import pallas as pl
from jax.experimental.pallas import tpu as pltpu

N = 4096
R = 512
C = 512


def _scan_kernel(x_ref, m_ref, o_ref, carry_ref):
    j = pl.program_id(1)

    @pl.when(j == 0)
    def _():
        carry_ref[...] = jnp.zeros_like(carry_ref)

    xm = jnp.where(m_ref[...], x_ref[...], 0.0)
    row = jax.lax.broadcasted_iota(jnp.int32, (C, C), 0)
    col = jax.lax.broadcasted_iota(jnp.int32, (C, C), 1)
    tri = (row <= col).astype(jnp.float32)
    cs = jax.lax.dot(xm, tri, precision=jax.lax.Precision.HIGHEST,
                     preferred_element_type=jnp.float32)
    out = cs + carry_ref[...]
    o_ref[...] = out
    carry_ref[...] = out[:, C - 1:C]


def kernel(x, mask):
    grid = (N // R, N // C)
    return pl.pallas_call(
        _scan_kernel,
        grid=grid,
        in_specs=[
            pl.BlockSpec((R, C), lambda i, j: (i, j)),
            pl.BlockSpec((R, C), lambda i, j: (i, j)),
        ],
        out_specs=pl.BlockSpec((R, C), lambda i, j: (i, j)),
        out_shape=jax.ShapeDtypeStruct((N, N), jnp.float32),
        scratch_shapes=[pltpu.VMEM((R, 1), jnp.float32)],
    )(x, mask)
